# Initial kernel scaffold; baseline (speedup 1.0000x reference)
import jax
import jax.numpy as jnp
from jax.experimental import pallas as pl


def kernel(x, edge_index, edge_attr, params):
    raise NotImplementedError("write your pallas kernel here")


# speedup vs baseline: 1.3228x; 1.3228x over previous
import jax
import jax.numpy as jnp
from jax.experimental import pallas as pl

N = 10000
E = 160000
EPS = 1e-5
BN_SCALE = 1.0 / (1.0 + EPS) ** 0.5

BE = 1600


def _dot(a, b):
    return jnp.dot(a, b, precision=jax.lax.Precision.HIGHEST)


def _mlp_msg_body(ea_ref, xs_ref, w1, w2, b2, w3, b3, w4, b4, w5, b5, out_ref):
    h = jax.nn.relu(_dot(ea_ref[...], w1[...]))
    h = jax.nn.relu(_dot(h, w2[...]) + b2[...])
    h = jax.nn.relu(_dot(h, w3[...]) + b3[...])
    h = jax.nn.relu(_dot(h, w4[...]) + b4[...])
    xs = xs_ref[...]
    acc = jnp.zeros((out_ref.shape[0], 256), jnp.float32)
    for i in range(7):
        z = _dot(h, w5[...][:, i * 256:(i + 1) * 256]) + b5[...][:, i * 256:(i + 1) * 256]
        acc = acc + xs[:, i:i + 1] * jax.nn.sigmoid(z)
    out_ref[...] = acc


def _fused_msg(ea1, xs, ws):
    full = lambda shape: pl.BlockSpec(shape, lambda i: (0, 0))
    return pl.pallas_call(
        _mlp_msg_body,
        grid=(E // BE,),
        in_specs=[
            pl.BlockSpec((BE, 4), lambda i: (i, 0)),
            pl.BlockSpec((BE, 8), lambda i: (i, 0)),
            full((4, 256)),
            full((256, 128)), full((1, 128)),
            full((128, 64)), full((1, 64)),
            full((64, 32)), full((1, 32)),
            full((32, 1792)), full((1, 1792)),
        ],
        out_specs=pl.BlockSpec((BE, 256), lambda i: (i, 0)),
        out_shape=jax.ShapeDtypeStruct((E, 256), jnp.float32),
    )(ea1, xs, *ws)


def kernel(x, edge_index, edge_attr, params):
    p = params
    src = edge_index[0]
    dst = edge_index[1]

    def fold(i):
        s = p['mg%d' % i] * BN_SCALE
        return p['mW%d' % i] * s[None, :], (p['mb%d' % i] * s + p['mbeta%d' % i])

    w1, c1 = fold(1)
    w2, c2 = fold(2)
    w3, c3 = fold(3)
    w4, c4 = fold(4)
    ww1 = jnp.concatenate([w1, c1[None, :]], axis=0)
    ws = (ww1, w2, c2[None, :], w3, c3[None, :], w4, c4[None, :],
          p['mW5'], p['mb5'][None, :])

    ea1 = jnp.concatenate([edge_attr, jnp.ones((E, 1), jnp.float32)], axis=1)
    xs = jnp.pad(x, ((0, 0), (0, 1)))[src]

    msg = _fused_msg(ea1, xs, ws)

    deg = jax.ops.segment_sum(jnp.ones((E,), jnp.float32), dst, num_segments=N)
    h = jax.ops.segment_sum(msg, dst, num_segments=N) / jnp.clip(deg, 1.0)[:, None]
    h = h + p['nn_bias']
    for i in range(5):
        a = _dot(h, p['tW%d' % i])
        c = _dot(h, p['pW%d' % i] - p['tW%d' % i]) + p['tb%d' % i] + p['pb%d' % i]
        m = jax.ops.segment_max(a[src], dst, num_segments=N)
        h = jnp.where(deg[:, None] > 0, m + c, 0.0)
    return h


# speedup vs baseline: 3.0805x; 2.3288x over previous
import functools

import jax
import jax.numpy as jnp
from jax import lax
from jax.experimental import pallas as pl
from jax.experimental.pallas import tpu as pltpu
from jax.experimental.pallas import tpu_sc as plsc

N = 10000
E = 160000
NP = 10240
NC, NS = 2, 16
NW = NC * NS
EPT = E // NS
NPT = NP // NS
NPW = NP // NW
EPS = 1e-5
BN_SCALE = 1.0 / (1.0 + EPS) ** 0.5
NEG = -3.0e38

BE = 1600
BN = 1000

_SCP = pltpu.CompilerParams(needs_layout_passes=False)


def _mesh():
    return plsc.VectorSubcoreMesh(core_axis_name="c", subcore_axis_name="s",
                                  num_cores=NC, num_subcores=NS)


def _dot(a, b):
    return jnp.dot(a, b, precision=jax.lax.Precision.HIGHEST)


def _mlp_msg_body(ea_ref, xs_ref, w1, w2, b2, w3, b3, w4, b4, w5, b5,
                  out0_ref, out1_ref):
    h = jax.nn.relu(_dot(ea_ref[...], w1[...]))
    h = jax.nn.relu(_dot(h, w2[...]) + b2[...])
    h = jax.nn.relu(_dot(h, w3[...]) + b3[...])
    h = jax.nn.relu(_dot(h, w4[...]) + b4[...])
    xs = xs_ref[...]
    acc = jnp.zeros((BE, 256), jnp.float32)
    for i in range(7):
        z = _dot(h, w5[...][:, i * 256:(i + 1) * 256]) \
            + b5[...][:, i * 256:(i + 1) * 256]
        acc = acc + xs[:, i:i + 1] * jax.nn.sigmoid(z)
    out0_ref[...] = acc[:, :128]
    out1_ref[...] = acc[:, 128:]


def _fused_msg(ea1, xs, ws):
    full = lambda shape: pl.BlockSpec(shape, lambda i: (0, 0))
    return pl.pallas_call(
        _mlp_msg_body,
        grid=(E // BE,),
        in_specs=[
            pl.BlockSpec((BE, 4), lambda i: (i, 0)),
            pl.BlockSpec((BE, 8), lambda i: (i, 0)),
            full((4, 256)),
            full((256, 128)), full((1, 128)),
            full((128, 64)), full((1, 64)),
            full((64, 32)), full((1, 32)),
            full((32, 1792)), full((1, 1792)),
        ],
        out_specs=[pl.BlockSpec((BE, 128), lambda i: (i, 0)),
                   pl.BlockSpec((BE, 128), lambda i: (i, 0))],
        out_shape=[jax.ShapeDtypeStruct((E, 128), jnp.float32),
                   jax.ShapeDtypeStruct((E, 128), jnp.float32)],
    )(ea1, xs, *ws)


def _xs_gather(x_wide, src):
    @functools.partial(
        pl.kernel,
        out_type=jax.ShapeDtypeStruct((E * 8,), jnp.float32),
        mesh=_mesh(),
        scratch_types=[
            pltpu.VMEM((128,), jnp.int32),
            pltpu.VMEM((128, 128), jnp.float32),
            pltpu.VMEM((5000 * 8,), jnp.float32),
            pltpu.SemaphoreType.DMA,
        ],
        compiler_params=_SCP,
    )
    def k(x_hbm, src_hbm, xs_hbm, idx_v, grows, rows_v, sem):
        w = lax.axis_index("c") * NS + lax.axis_index("s")
        e0 = w * 5000
        riota = lax.iota(jnp.int32, 16)

        def body(jj, _):
            boff = jnp.minimum(jj * 128, 5000 - 128)
            pltpu.sync_copy(src_hbm.at[pl.ds(e0 + boff, 128)], idx_v)
            pltpu.async_copy(x_hbm.at[idx_v], grows, sem).wait()
            for g in range(8):
                ridx = riota + g * 16
                for kcol in range(8):
                    vals = plsc.load_gather(
                        grows, [ridx, jnp.full((16,), kcol, jnp.int32)])
                    plsc.store_scatter(
                        rows_v, [(boff + ridx) * 8 + kcol], vals)
            return 0

        lax.fori_loop(0, 40, body, 0)
        pltpu.sync_copy(rows_v, xs_hbm.at[pl.ds(e0 * 8, 5000 * 8)])

    return k(x_wide, src)


def _sort_csr(dst, src):
    @functools.partial(
        pl.kernel,
        out_type=(
            jax.ShapeDtypeStruct((E + 128,), jnp.int32),
            jax.ShapeDtypeStruct((NP + 16,), jnp.int32),
            jax.ShapeDtypeStruct((NP,), jnp.float32),
        ),
        mesh=_mesh(),
        scratch_types=[
            pltpu.VMEM((EPT,), jnp.int32),
            pltpu.VMEM((EPT,), jnp.int32),
            pltpu.VMEM((NP,), jnp.int32),
            pltpu.VMEM((NS * NPT,), jnp.int32),
            pltpu.VMEM((NPT,), jnp.int32),
            pltpu.VMEM((NPT,), jnp.int32),
            pltpu.VMEM((NPT,), jnp.float32),
            pltpu.VMEM((NPT,), jnp.int32),
            pltpu.VMEM((16,), jnp.int32),
            pltpu.VMEM((NS * 16,), jnp.int32),
            pltpu.VMEM((128,), jnp.int32),
            pltpu.VMEM((128,), jnp.int32),
            pltpu.VMEM_SHARED((NS * NP,), jnp.int32),
            pltpu.VMEM_SHARED((NS * 16,), jnp.int32),
        ],
        compiler_params=_SCP,
    )
    def k(dst_hbm, src_hbm, srcp_hbm, base_hbm, deg_hbm,
          dstb, srcb, hist_v, gcols, tot_v, base_v, deg_v, row_v, tbuf,
          sums2, posb, srow, g_sp, sums_sp):
        cid = lax.axis_index("c")
        t = lax.axis_index("s")

        @pl.when(cid == 0)
        def _():
            pltpu.sync_copy(dst_hbm.at[pl.ds(t * EPT, EPT)], dstb)

            def zero(i, _):
                hist_v[pl.ds(i * 16, 16)] = jnp.zeros((16,), jnp.int32)
                return 0
            lax.fori_loop(0, NP // 16, zero, 0)

            def hbody(i, _):
                d = dstb[pl.ds(i * 16, 16)]
                occ, last = plsc.scan_count(d)
                plsc.addupdate_scatter(hist_v, [d], occ, mask=last)
                return 0
            lax.fori_loop(0, EPT // 16, hbody, 0)
            pltpu.sync_copy(hist_v, g_sp.at[pl.ds(t * NP, NP)])
            plsc.subcore_barrier()

            for r in range(NS):
                pltpu.sync_copy(g_sp.at[pl.ds(r * NP + t * NPT, NPT)],
                                gcols.at[pl.ds(r * NPT, NPT)])
            def tbody(i, _):
                s = jnp.zeros((16,), jnp.int32)
                for r in range(NS):
                    s = s + gcols[pl.ds(r * NPT + i * 16, 16)]
                tot_v[pl.ds(i * 16, 16)] = s
                return 0
            lax.fori_loop(0, NPT // 16, tbody, 0)

            def sbody(i, acc):
                return acc + tot_v[pl.ds(i * 16, 16)]
            rs = lax.fori_loop(0, NPT // 16, sbody, jnp.zeros((16,), jnp.int32))
            range_sum = jnp.sum(rs)
            tbuf[...] = jnp.full((16,), 0, jnp.int32) + range_sum
            pltpu.sync_copy(tbuf, sums_sp.at[pl.ds(t * 16, 16)])
            plsc.subcore_barrier()

            pltpu.sync_copy(sums_sp, sums2)
            lanes = lax.iota(jnp.int32, 16)
            svec = plsc.load_gather(sums2, [lanes * 16])
            base0 = jnp.sum(jnp.where(lanes < t, svec, 0))

            def cbody(i, carry):
                v = tot_v[pl.ds(i * 16, 16)]
                inc = plsc.cumsum(v)
                base_v[pl.ds(i * 16, 16)] = carry + inc - v
                deg_v[pl.ds(i * 16, 16)] = v.astype(jnp.float32)
                return carry + jnp.max(inc)
            lax.fori_loop(0, NPT // 16, cbody, base0)

            pltpu.sync_copy(base_v, base_hbm.at[pl.ds(t * NPT, NPT)])
            pltpu.sync_copy(deg_v, deg_hbm.at[pl.ds(t * NPT, NPT)])

            @pl.when(t == NS - 1)
            def _():
                tbuf[...] = jnp.full((16,), E, jnp.int32)
                pltpu.sync_copy(tbuf, base_hbm.at[pl.ds(NP, 16)])

            def copy_base(i, _):
                row_v[pl.ds(i * 16, 16)] = base_v[pl.ds(i * 16, 16)]
                return 0
            lax.fori_loop(0, NPT // 16, copy_base, 0)
            for r in range(NS):
                pltpu.sync_copy(row_v, g_sp.at[pl.ds(r * NP + t * NPT, NPT)])
                def add_r(i, _):
                    row_v[pl.ds(i * 16, 16)] = (
                        row_v[pl.ds(i * 16, 16)]
                        + gcols[pl.ds(r * NPT + i * 16, 16)])
                    return 0
                if r < NS - 1:
                    lax.fori_loop(0, NPT // 16, add_r, 0)
            plsc.subcore_barrier()

            pltpu.sync_copy(g_sp.at[pl.ds(t * NP, NP)], hist_v)
            pltpu.sync_copy(src_hbm.at[pl.ds(t * EPT, EPT)], srcb)

            def chunk(j, _):
                for g in range(8):
                    d = dstb[pl.ds(j * 128 + g * 16, 16)]
                    occ, last = plsc.scan_count(d)
                    cur = plsc.load_gather(hist_v, [d])
                    posb[pl.ds(g * 16, 16)] = cur + occ - 1
                    plsc.addupdate_scatter(hist_v, [d], occ, mask=last)
                    srow[pl.ds(g * 16, 16)] = srcb[pl.ds(j * 128 + g * 16, 16)]
                pltpu.sync_copy(srow, srcp_hbm.at[posb])
                return 0
            lax.fori_loop(0, EPT // 128, chunk, 0)

            if EPT % 128:
                j0 = (EPT // 128) * 128
                d = dstb[pl.ds(j0, 16)]
                occ, last = plsc.scan_count(d)
                cur = plsc.load_gather(hist_v, [d])
                posb[pl.ds(0, 16)] = cur + occ - 1
                plsc.addupdate_scatter(hist_v, [d], occ, mask=last)
                srow[pl.ds(0, 16)] = srcb[pl.ds(j0, 16)]
                def padp(g, _):
                    posb[pl.ds(16 + g * 16, 16)] = jnp.full((16,), E, jnp.int32)
                    return 0
                lax.fori_loop(0, 7, padp, 0)
                pltpu.sync_copy(srow, srcp_hbm.at[posb])

    return k(dst, src)


def _seg_sum(msg0, msg1, dst, zeros_np):
    @functools.partial(
        pl.kernel,
        out_type=(jax.ShapeDtypeStruct((NP, 128), jnp.float32),
                  jax.ShapeDtypeStruct((NP, 128), jnp.float32)),
        mesh=_mesh(),
        scratch_types=[
            pltpu.VMEM((128, 128), jnp.float32),
            pltpu.VMEM((128,), jnp.int32),
            pltpu.VMEM((16,), jnp.int32),
            pltpu.VMEM((128, 128), jnp.float32),
            pltpu.VMEM_SHARED((NP, 128), jnp.float32),
        ],
        compiler_params=_SCP,
    )
    def k(m0_hbm, m1_hbm, dst_hbm, z_hbm, out0_hbm, out1_hbm,
          mbuf, didx, didx16, wb, s_sp):
        cid = lax.axis_index("c")
        t = lax.axis_index("s")
        m_hbm = [m0_hbm, m1_hbm]
        out_hbm = [out0_hbm, out1_hbm]

        @pl.when(t == 0)
        def _():
            pltpu.sync_copy(z_hbm, s_sp)
        plsc.subcore_barrier()

        for c in range(NC):
            @pl.when(cid == c)
            def _(c=c):
                def body(j, _):
                    e0 = t * EPT + j * 128
                    pltpu.sync_copy(m_hbm[c].at[pl.ds(e0, 128), :], mbuf)
                    pltpu.sync_copy(dst_hbm.at[pl.ds(e0, 128)], didx)
                    pltpu.sync_copy(mbuf, s_sp.at[didx], add=True)
                    return 0
                lax.fori_loop(0, EPT // 128, body, 0)
                e0t = t * EPT + (EPT // 128) * 128
                pltpu.sync_copy(m_hbm[c].at[pl.ds(e0t, 16), :],
                                mbuf.at[pl.ds(0, 16), :])
                pltpu.sync_copy(dst_hbm.at[pl.ds(e0t, 16)], didx16)
                pltpu.sync_copy(mbuf.at[pl.ds(0, 16), :],
                                s_sp.at[didx16], add=True)
        plsc.subcore_barrier()

        n0 = t * 640
        iota16 = lax.iota(jnp.int32, 16)
        def wb_chunk(kc, _):
            r0 = n0 + kc * 128
            for g in range(8):
                didx[pl.ds(g * 16, 16)] = r0 + g * 16 + iota16
            pltpu.sync_copy(s_sp.at[didx], wb)
            for c in range(NC):
                @pl.when(cid == c)
                def _(c=c):
                    pltpu.sync_copy(wb, out_hbm[c].at[pl.ds(r0, 128), :])
            return 0
        lax.fori_loop(0, 5, wb_chunk, 0)

    return k(msg0, msg1, dst, zeros_np)


def _seg_max(a, srcp, base, b):
    nb = b // 16

    @functools.partial(
        pl.kernel,
        out_type=jax.ShapeDtypeStruct((NP * b,), jnp.float32),
        mesh=_mesh(),
        scratch_types=[
            pltpu.VMEM((336,), jnp.int32),
            pltpu.VMEM((128,), jnp.int32),
            pltpu.VMEM((128, 128), jnp.float32),
            pltpu.VMEM((NPW * b,), jnp.float32),
            pltpu.SemaphoreType.DMA,
        ],
        compiler_params=_SCP,
    )
    def k(a_hbm, srcp_hbm, base_hbm, m_hbm, base_sm, sidx, rows, outb, sem):
        w = lax.axis_index("c") * NS + lax.axis_index("s")
        n0 = w * NPW

        def bval(i):
            return base_sm[pl.ds(i, 16)][0]

        pltpu.sync_copy(base_hbm.at[pl.ds(n0, 328)],
                        base_sm.at[pl.ds(0, 328)])
        sp = bval(0)
        ep = bval(NPW)

        def zero(i, _):
            outb[pl.ds(i * 16, 16)] = jnp.full((16,), NEG, jnp.float32)
            return 0
        lax.fori_loop(0, NPW * b // 16, zero, 0)

        neg = tuple(jnp.full((16,), NEG, jnp.float32) for _ in range(nb))
        iota16 = lax.iota(jnp.int32, 16)
        cols = [iota16 + q * 16 for q in range(nb)]

        def chunk(cg, carry):
            n, p, acc0 = carry
            pltpu.sync_copy(srcp_hbm.at[pl.ds(cg * 128, 128)], sidx)
            pltpu.async_copy(a_hbm.at[sidx], rows, sem).wait()
            hi = jnp.minimum(ep, (cg + 1) * 128)
            clo = cg * 128

            def cond(st):
                return st[1] < hi

            def step(st):
                n_, p_, acc = st
                e_n = bval(n_ - n0 + 1)
                seg_hi = jnp.minimum(e_n, hi)

                def pos_body(pp, acc_):
                    rvec = jnp.full((16,), pp - clo, jnp.int32)
                    return tuple(
                        jnp.maximum(acc_[q],
                                    plsc.load_gather(rows, [rvec, cols[q]]))
                        for q in range(nb))
                acc = lax.fori_loop(p_, seg_hi, pos_body, acc)

                flushed = e_n <= hi

                @pl.when(flushed)
                def _():
                    o = (n_ - n0) * b
                    for q in range(nb):
                        outb[pl.ds(o + q * 16, 16)] = acc[q]

                acc = tuple(
                    jnp.where(flushed, neg[q], acc[q]) for q in range(nb))
                n_ = jnp.where(flushed, n_ + 1, n_)
                return (n_, seg_hi, acc)

            st = lax.while_loop(cond, step, (n, jnp.maximum(p, clo), acc0))
            return st

        cg0 = sp // 128
        cg1 = (ep + 127) // 128
        lax.fori_loop(cg0, cg1, chunk, (n0, sp, neg))

        pltpu.sync_copy(outb, m_hbm.at[pl.ds(n0 * b, NPW * b)])

    return k(a, srcp, base)


def _layer0(s0, s1, deg2, nn_bias, tw, cw, cb):
    a_dim = 256
    b_dim = tw.shape[1]

    def body(s0_ref, s1_ref, d_ref, bias, twr, cwr, cbr, a_out, c_out):
        d = d_ref[...]
        s = jnp.concatenate([s0_ref[...], s1_ref[...]], axis=1)
        h = s / jnp.maximum(d, 1.0) + bias[...]
        a = _dot(h, twr[...])
        if b_dim < 128:
            a = jnp.concatenate(
                [a, jnp.zeros((BN, 128 - b_dim), jnp.float32)], axis=1)
        a_out[...] = a
        c_out[...] = _dot(h, cwr[...]) + cbr[...]

    full = lambda shape: pl.BlockSpec(shape, lambda i: (0, 0))
    return pl.pallas_call(
        body,
        grid=(N // BN,),
        in_specs=[
            pl.BlockSpec((BN, 128), lambda i: (i, 0)),
            pl.BlockSpec((BN, 128), lambda i: (i, 0)),
            pl.BlockSpec((BN, 1), lambda i: (i, 0)),
            full((1, a_dim)), full((a_dim, b_dim)), full((a_dim, b_dim)),
            full((1, b_dim)),
        ],
        out_specs=[pl.BlockSpec((BN, 128), lambda i: (i, 0)),
                   pl.BlockSpec((BN, b_dim), lambda i: (i, 0))],
        out_shape=[jax.ShapeDtypeStruct((N, 128), jnp.float32),
                   jax.ShapeDtypeStruct((N, b_dim), jnp.float32)],
    )(s0, s1, deg2, nn_bias, tw, cw, cb)


def _layer(m, c, deg2, tw, cw, cb):
    a_dim = tw.shape[0]
    b_dim = tw.shape[1]

    def body(m_ref, c_ref, d_ref, twr, cwr, cbr, a_out, c_out):
        h = jnp.where(d_ref[...] > 0, m_ref[...] + c_ref[...], 0.0)
        a = _dot(h, twr[...])
        if b_dim < 128:
            a = jnp.concatenate(
                [a, jnp.zeros((BN, 128 - b_dim), jnp.float32)], axis=1)
        a_out[...] = a
        c_out[...] = _dot(h, cwr[...]) + cbr[...]

    full = lambda shape: pl.BlockSpec(shape, lambda i: (0, 0))
    return pl.pallas_call(
        body,
        grid=(N // BN,),
        in_specs=[
            pl.BlockSpec((BN, a_dim), lambda i: (i, 0)),
            pl.BlockSpec((BN, a_dim), lambda i: (i, 0)),
            pl.BlockSpec((BN, 1), lambda i: (i, 0)),
            full((a_dim, b_dim)), full((a_dim, b_dim)), full((1, b_dim)),
        ],
        out_specs=[pl.BlockSpec((BN, 128), lambda i: (i, 0)),
                   pl.BlockSpec((BN, b_dim), lambda i: (i, 0))],
        out_shape=[jax.ShapeDtypeStruct((N, 128), jnp.float32),
                   jax.ShapeDtypeStruct((N, b_dim), jnp.float32)],
    )(m, c, deg2, tw, cw, cb)


def _final(m, c, deg2):
    b_dim = m.shape[1]

    def body(m_ref, c_ref, d_ref, out):
        out[...] = jnp.where(d_ref[...] > 0, m_ref[...] + c_ref[...], 0.0)

    return pl.pallas_call(
        body,
        grid=(N // BN,),
        in_specs=[
            pl.BlockSpec((BN, b_dim), lambda i: (i, 0)),
            pl.BlockSpec((BN, b_dim), lambda i: (i, 0)),
            pl.BlockSpec((BN, 1), lambda i: (i, 0)),
        ],
        out_specs=pl.BlockSpec((BN, b_dim), lambda i: (i, 0)),
        out_shape=jax.ShapeDtypeStruct((N, b_dim), jnp.float32),
    )(m, c, deg2)


def kernel(x, edge_index, edge_attr, params):
    p = params
    src = edge_index[0]
    dst = edge_index[1]

    def fold(i):
        s = p['mg%d' % i] * BN_SCALE
        return p['mW%d' % i] * s[None, :], (p['mb%d' % i] * s + p['mbeta%d' % i])

    w1, c1 = fold(1)
    w2, c2 = fold(2)
    w3, c3 = fold(3)
    w4, c4 = fold(4)
    ww1 = jnp.concatenate([w1, c1[None, :]], axis=0)
    ws = (ww1, w2, c2[None, :], w3, c3[None, :], w4, c4[None, :],
          p['mW5'], p['mb5'][None, :])

    ea1 = jnp.concatenate([edge_attr, jnp.ones((E, 1), jnp.float32)], axis=1)

    x_wide = jnp.pad(x, ((0, 0), (0, 121)))
    xs = _xs_gather(x_wide, src).reshape(E, 8)
    srcp, base, deg = _sort_csr(dst, src)
    msg0, msg1 = _fused_msg(ea1, xs, ws)
    s0, s1 = _seg_sum(msg0, msg1, dst, jnp.zeros((NP, 128), jnp.float32))
    s0, s1 = s0[:N], s1[:N]
    deg2 = deg[:N].reshape(N, 1)

    dims = [(256, 128), (128, 64), (64, 32), (32, 16), (16, 16)]
    a_arr = c_arr = None
    for i, (ad, bd) in enumerate(dims):
        tw = p['tW%d' % i]
        cw = p['pW%d' % i] - tw
        cb = (p['tb%d' % i] + p['pb%d' % i])[None, :]
        if i == 0:
            a_arr, c_arr = _layer0(s0, s1, deg2, p['nn_bias'][None, :], tw, cw, cb)
        else:
            a_arr, c_arr = _layer(m_arr, c_arr, deg2, tw, cw, cb)
        m_flat = _seg_max(a_arr, srcp, base, bd)
        m_arr = m_flat.reshape(NP, bd)[:N]
    return _final(m_arr, c_arr, deg2)


# speedup vs baseline: 5.4054x; 1.7547x over previous
import functools

import jax
import jax.numpy as jnp
from jax import lax
from jax.experimental import pallas as pl
from jax.experimental.pallas import tpu as pltpu
from jax.experimental.pallas import tpu_sc as plsc

N = 10000
E = 160000
NP = 10240
NC, NS = 2, 16
NW = NC * NS
EPT = E // NS
NPT = NP // NS
NPW = NP // NW
EPS = 1e-5
BN_SCALE = 1.0 / (1.0 + EPS) ** 0.5
NEG = -3.0e38

BE = 1600
BN = 1000

_SCP = pltpu.CompilerParams(needs_layout_passes=False)


def _mesh():
    return plsc.VectorSubcoreMesh(core_axis_name="c", subcore_axis_name="s",
                                  num_cores=NC, num_subcores=NS)


def _dot(a, b):
    return jnp.dot(a, b, precision=jax.lax.Precision.HIGHEST)


def _dot_hi(a, b):
    return jnp.dot(a, b, precision=jax.lax.Precision.DEFAULT)


def _mlp_msg_body(ea_ref, xs_ref, w1, w2, b2, w3, b3, w4, b4, w5, b5,
                  out0_ref, out1_ref):
    h = jax.nn.relu(_dot_hi(ea_ref[...], w1[...]))
    h = jax.nn.relu(_dot_hi(h, w2[...]) + b2[...])
    h = jax.nn.relu(_dot_hi(h, w3[...]) + b3[...])
    h = jax.nn.relu(_dot_hi(h, w4[...]) + b4[...])
    xs = xs_ref[...]
    acc = jnp.zeros((BE, 256), jnp.float32)
    for i in range(7):
        z = _dot_hi(h, w5[...][:, i * 256:(i + 1) * 256]) \
            + b5[...][:, i * 256:(i + 1) * 256]
        acc = acc + xs[:, i:i + 1] * jax.nn.sigmoid(z)
    out0_ref[...] = acc[:, :128]
    out1_ref[...] = acc[:, 128:]


def _fused_msg(ea1, xs, ws):
    full = lambda shape: pl.BlockSpec(shape, lambda i: (0, 0))
    return pl.pallas_call(
        _mlp_msg_body,
        grid=(E // BE,),
        in_specs=[
            pl.BlockSpec((BE, 4), lambda i: (i, 0)),
            pl.BlockSpec((BE, 8), lambda i: (i, 0)),
            full((4, 256)),
            full((256, 128)), full((1, 128)),
            full((128, 64)), full((1, 64)),
            full((64, 32)), full((1, 32)),
            full((32, 1792)), full((1, 1792)),
        ],
        out_specs=[pl.BlockSpec((BE, 128), lambda i: (i, 0)),
                   pl.BlockSpec((BE, 128), lambda i: (i, 0))],
        out_shape=[jax.ShapeDtypeStruct((E, 128), jnp.float32),
                   jax.ShapeDtypeStruct((E, 128), jnp.float32)],
    )(ea1, xs, *ws)


def _xs_gather(x_wide, src):
    @functools.partial(
        pl.kernel,
        out_type=jax.ShapeDtypeStruct((E * 8,), jnp.float32),
        mesh=_mesh(),
        scratch_types=[
            pltpu.VMEM((128,), jnp.int32),
            pltpu.VMEM((128, 128), jnp.float32),
            pltpu.VMEM((5000 * 8,), jnp.float32),
            pltpu.SemaphoreType.DMA,
        ],
        compiler_params=_SCP,
    )
    def k(x_hbm, src_hbm, xs_hbm, idx_v, grows, rows_v, sem):
        w = lax.axis_index("c") * NS + lax.axis_index("s")
        e0 = w * 5000
        riota = lax.iota(jnp.int32, 16)

        def body(jj, _):
            boff = jnp.minimum(jj * 128, 5000 - 128)
            pltpu.sync_copy(src_hbm.at[pl.ds(e0 + boff, 128)], idx_v)
            pltpu.async_copy(x_hbm.at[idx_v], grows, sem).wait()
            for g in range(8):
                ridx = riota + g * 16
                for kcol in range(8):
                    vals = plsc.load_gather(
                        grows, [ridx, jnp.full((16,), kcol, jnp.int32)])
                    plsc.store_scatter(
                        rows_v, [(boff + ridx) * 8 + kcol], vals)
            return 0

        lax.fori_loop(0, 40, body, 0)
        pltpu.sync_copy(rows_v, xs_hbm.at[pl.ds(e0 * 8, 5000 * 8)])

    return k(x_wide, src)


def _sort_csr(dst, src):
    @functools.partial(
        pl.kernel,
        out_type=(
            jax.ShapeDtypeStruct((E + 128,), jnp.int32),
            jax.ShapeDtypeStruct((NP + 16,), jnp.int32),
            jax.ShapeDtypeStruct((NP,), jnp.float32),
        ),
        mesh=_mesh(),
        scratch_types=[
            pltpu.VMEM((EPT,), jnp.int32),
            pltpu.VMEM((EPT,), jnp.int32),
            pltpu.VMEM((NP,), jnp.int32),
            pltpu.VMEM((NS * NPT,), jnp.int32),
            pltpu.VMEM((NPT,), jnp.int32),
            pltpu.VMEM((NPT,), jnp.int32),
            pltpu.VMEM((NPT,), jnp.float32),
            pltpu.VMEM((NPT,), jnp.int32),
            pltpu.VMEM((16,), jnp.int32),
            pltpu.VMEM((NS * 16,), jnp.int32),
            pltpu.VMEM((128,), jnp.int32),
            pltpu.VMEM((128,), jnp.int32),
            pltpu.VMEM_SHARED((NS * NP,), jnp.int32),
            pltpu.VMEM_SHARED((NS * 16,), jnp.int32),
        ],
        compiler_params=_SCP,
    )
    def k(dst_hbm, src_hbm, srcp_hbm, base_hbm, deg_hbm,
          dstb, srcb, hist_v, gcols, tot_v, base_v, deg_v, row_v, tbuf,
          sums2, posb, srow, g_sp, sums_sp):
        cid = lax.axis_index("c")
        t = lax.axis_index("s")

        @pl.when(cid == 0)
        def _():
            pltpu.sync_copy(dst_hbm.at[pl.ds(t * EPT, EPT)], dstb)

            def zero(i, _):
                hist_v[pl.ds(i * 16, 16)] = jnp.zeros((16,), jnp.int32)
                return 0
            lax.fori_loop(0, NP // 16, zero, 0)

            def hbody(i, _):
                d = dstb[pl.ds(i * 16, 16)]
                occ, last = plsc.scan_count(d)
                plsc.addupdate_scatter(hist_v, [d], occ, mask=last)
                return 0
            lax.fori_loop(0, EPT // 16, hbody, 0)
            pltpu.sync_copy(hist_v, g_sp.at[pl.ds(t * NP, NP)])
            plsc.subcore_barrier()

            for r in range(NS):
                pltpu.sync_copy(g_sp.at[pl.ds(r * NP + t * NPT, NPT)],
                                gcols.at[pl.ds(r * NPT, NPT)])
            def tbody(i, _):
                s = jnp.zeros((16,), jnp.int32)
                for r in range(NS):
                    s = s + gcols[pl.ds(r * NPT + i * 16, 16)]
                tot_v[pl.ds(i * 16, 16)] = s
                return 0
            lax.fori_loop(0, NPT // 16, tbody, 0)

            def sbody(i, acc):
                return acc + tot_v[pl.ds(i * 16, 16)]
            rs = lax.fori_loop(0, NPT // 16, sbody, jnp.zeros((16,), jnp.int32))
            range_sum = jnp.sum(rs)
            tbuf[...] = jnp.full((16,), 0, jnp.int32) + range_sum
            pltpu.sync_copy(tbuf, sums_sp.at[pl.ds(t * 16, 16)])
            plsc.subcore_barrier()

            pltpu.sync_copy(sums_sp, sums2)
            lanes = lax.iota(jnp.int32, 16)
            svec = plsc.load_gather(sums2, [lanes * 16])
            base0 = jnp.sum(jnp.where(lanes < t, svec, 0))

            def cbody(i, carry):
                v = tot_v[pl.ds(i * 16, 16)]
                inc = plsc.cumsum(v)
                base_v[pl.ds(i * 16, 16)] = carry + inc - v
                deg_v[pl.ds(i * 16, 16)] = v.astype(jnp.float32)
                return carry + jnp.max(inc)
            lax.fori_loop(0, NPT // 16, cbody, base0)

            pltpu.sync_copy(base_v, base_hbm.at[pl.ds(t * NPT, NPT)])
            pltpu.sync_copy(deg_v, deg_hbm.at[pl.ds(t * NPT, NPT)])

            @pl.when(t == NS - 1)
            def _():
                tbuf[...] = jnp.full((16,), E, jnp.int32)
                pltpu.sync_copy(tbuf, base_hbm.at[pl.ds(NP, 16)])

            def copy_base(i, _):
                row_v[pl.ds(i * 16, 16)] = base_v[pl.ds(i * 16, 16)]
                return 0
            lax.fori_loop(0, NPT // 16, copy_base, 0)
            for r in range(NS):
                pltpu.sync_copy(row_v, g_sp.at[pl.ds(r * NP + t * NPT, NPT)])
                def add_r(i, _):
                    row_v[pl.ds(i * 16, 16)] = (
                        row_v[pl.ds(i * 16, 16)]
                        + gcols[pl.ds(r * NPT + i * 16, 16)])
                    return 0
                if r < NS - 1:
                    lax.fori_loop(0, NPT // 16, add_r, 0)
            plsc.subcore_barrier()

            pltpu.sync_copy(g_sp.at[pl.ds(t * NP, NP)], hist_v)
            pltpu.sync_copy(src_hbm.at[pl.ds(t * EPT, EPT)], srcb)

            def chunk(j, _):
                for g in range(8):
                    d = dstb[pl.ds(j * 128 + g * 16, 16)]
                    occ, last = plsc.scan_count(d)
                    cur = plsc.load_gather(hist_v, [d])
                    posb[pl.ds(g * 16, 16)] = cur + occ - 1
                    plsc.addupdate_scatter(hist_v, [d], occ, mask=last)
                    srow[pl.ds(g * 16, 16)] = srcb[pl.ds(j * 128 + g * 16, 16)]
                pltpu.sync_copy(srow, srcp_hbm.at[posb])
                return 0
            lax.fori_loop(0, EPT // 128, chunk, 0)

            if EPT % 128:
                j0 = (EPT // 128) * 128
                d = dstb[pl.ds(j0, 16)]
                occ, last = plsc.scan_count(d)
                cur = plsc.load_gather(hist_v, [d])
                posb[pl.ds(0, 16)] = cur + occ - 1
                plsc.addupdate_scatter(hist_v, [d], occ, mask=last)
                srow[pl.ds(0, 16)] = srcb[pl.ds(j0, 16)]
                def padp(g, _):
                    posb[pl.ds(16 + g * 16, 16)] = jnp.full((16,), E, jnp.int32)
                    return 0
                lax.fori_loop(0, 7, padp, 0)
                pltpu.sync_copy(srow, srcp_hbm.at[posb])

    return k(dst, src)


def _seg_sum(msg0, msg1, dst, zeros_np):
    @functools.partial(
        pl.kernel,
        out_type=(jax.ShapeDtypeStruct((NP, 128), jnp.float32),
                  jax.ShapeDtypeStruct((NP, 128), jnp.float32)),
        mesh=_mesh(),
        scratch_types=[
            pltpu.VMEM((128, 128), jnp.float32),
            pltpu.VMEM((128,), jnp.int32),
            pltpu.VMEM((16,), jnp.int32),
            pltpu.VMEM((128, 128), jnp.float32),
            pltpu.VMEM_SHARED((NP, 128), jnp.float32),
        ],
        compiler_params=_SCP,
    )
    def k(m0_hbm, m1_hbm, dst_hbm, z_hbm, out0_hbm, out1_hbm,
          mbuf, didx, didx16, wb, s_sp):
        cid = lax.axis_index("c")
        t = lax.axis_index("s")
        m_hbm = [m0_hbm, m1_hbm]
        out_hbm = [out0_hbm, out1_hbm]

        @pl.when(t == 0)
        def _():
            pltpu.sync_copy(z_hbm, s_sp)
        plsc.subcore_barrier()

        for c in range(NC):
            @pl.when(cid == c)
            def _(c=c):
                def body(j, _):
                    e0 = t * EPT + j * 128
                    pltpu.sync_copy(m_hbm[c].at[pl.ds(e0, 128), :], mbuf)
                    pltpu.sync_copy(dst_hbm.at[pl.ds(e0, 128)], didx)
                    pltpu.sync_copy(mbuf, s_sp.at[didx], add=True)
                    return 0
                lax.fori_loop(0, EPT // 128, body, 0)
                e0t = t * EPT + (EPT // 128) * 128
                pltpu.sync_copy(m_hbm[c].at[pl.ds(e0t, 16), :],
                                mbuf.at[pl.ds(0, 16), :])
                pltpu.sync_copy(dst_hbm.at[pl.ds(e0t, 16)], didx16)
                pltpu.sync_copy(mbuf.at[pl.ds(0, 16), :],
                                s_sp.at[didx16], add=True)
        plsc.subcore_barrier()

        n0 = t * 640
        iota16 = lax.iota(jnp.int32, 16)
        def wb_chunk(kc, _):
            r0 = n0 + kc * 128
            for g in range(8):
                didx[pl.ds(g * 16, 16)] = r0 + g * 16 + iota16
            pltpu.sync_copy(s_sp.at[didx], wb)
            for c in range(NC):
                @pl.when(cid == c)
                def _(c=c):
                    pltpu.sync_copy(wb, out_hbm[c].at[pl.ds(r0, 128), :])
            return 0
        lax.fori_loop(0, 5, wb_chunk, 0)

    return k(msg0, msg1, dst, zeros_np)


def _seg_max(a, srcp, base, b):
    nb = b // 16

    @functools.partial(
        pl.kernel,
        out_type=jax.ShapeDtypeStruct((NP * b,), jnp.float32),
        mesh=_mesh(),
        scratch_types=[
            pltpu.VMEM((336,), jnp.int32),
            pltpu.VMEM((128,), jnp.int32),
            pltpu.VMEM((128, 128), jnp.float32),
            pltpu.VMEM((NPW * b,), jnp.float32),
            pltpu.SemaphoreType.DMA,
        ],
        compiler_params=_SCP,
    )
    def k(a_hbm, srcp_hbm, base_hbm, m_hbm, base_sm, sidx, rows, outb, sem):
        w = lax.axis_index("c") * NS + lax.axis_index("s")
        n0 = w * NPW

        def bval(i):
            return base_sm[pl.ds(i, 16)][0]

        pltpu.sync_copy(base_hbm.at[pl.ds(n0, 328)],
                        base_sm.at[pl.ds(0, 328)])
        sp = bval(0)
        ep = bval(NPW)

        def zero(i, _):
            outb[pl.ds(i * 16, 16)] = jnp.full((16,), NEG, jnp.float32)
            return 0
        lax.fori_loop(0, NPW * b // 16, zero, 0)

        neg = tuple(jnp.full((16,), NEG, jnp.float32) for _ in range(nb))
        iota16 = lax.iota(jnp.int32, 16)
        cols = [iota16 + q * 16 for q in range(nb)]

        def chunk(cg, carry):
            n, p, acc0 = carry
            pltpu.sync_copy(srcp_hbm.at[pl.ds(cg * 128, 128)], sidx)
            pltpu.async_copy(a_hbm.at[sidx], rows, sem).wait()
            hi = jnp.minimum(ep, (cg + 1) * 128)
            clo = cg * 128

            def cond(st):
                return st[1] < hi

            def step(st):
                n_, p_, acc = st
                e_n = bval(n_ - n0 + 1)
                seg_hi = jnp.minimum(e_n, hi)

                def pos_body(pp, acc_):
                    rvec = jnp.full((16,), pp - clo, jnp.int32)
                    return tuple(
                        jnp.maximum(acc_[q],
                                    plsc.load_gather(rows, [rvec, cols[q]]))
                        for q in range(nb))
                acc = lax.fori_loop(p_, seg_hi, pos_body, acc)

                flushed = e_n <= hi

                @pl.when(flushed)
                def _():
                    o = (n_ - n0) * b
                    for q in range(nb):
                        outb[pl.ds(o + q * 16, 16)] = acc[q]

                acc = tuple(
                    jnp.where(flushed, neg[q], acc[q]) for q in range(nb))
                n_ = jnp.where(flushed, n_ + 1, n_)
                return (n_, seg_hi, acc)

            st = lax.while_loop(cond, step, (n, jnp.maximum(p, clo), acc0))
            return st

        cg0 = sp // 128
        cg1 = (ep + 127) // 128
        lax.fori_loop(cg0, cg1, chunk, (n0, sp, neg))

        pltpu.sync_copy(outb, m_hbm.at[pl.ds(n0 * b, NPW * b)])

    return k(a, srcp, base)


def _layer0(s0, s1, deg2, nn_bias, tw, cw, cb):
    a_dim = 256
    b_dim = tw.shape[1]

    def body(s0_ref, s1_ref, d_ref, bias, twr, cwr, cbr, a_out, c_out):
        d = d_ref[...]
        s = jnp.concatenate([s0_ref[...], s1_ref[...]], axis=1)
        h = s / jnp.maximum(d, 1.0) + bias[...]
        a = _dot(h, twr[...])
        if b_dim < 128:
            a = jnp.concatenate(
                [a, jnp.zeros((BN, 128 - b_dim), jnp.float32)], axis=1)
        a_out[...] = a
        c_out[...] = _dot(h, cwr[...]) + cbr[...]

    full = lambda shape: pl.BlockSpec(shape, lambda i: (0, 0))
    return pl.pallas_call(
        body,
        grid=(N // BN,),
        in_specs=[
            pl.BlockSpec((BN, 128), lambda i: (i, 0)),
            pl.BlockSpec((BN, 128), lambda i: (i, 0)),
            pl.BlockSpec((BN, 1), lambda i: (i, 0)),
            full((1, a_dim)), full((a_dim, b_dim)), full((a_dim, b_dim)),
            full((1, b_dim)),
        ],
        out_specs=[pl.BlockSpec((BN, 128), lambda i: (i, 0)),
                   pl.BlockSpec((BN, b_dim), lambda i: (i, 0))],
        out_shape=[jax.ShapeDtypeStruct((N, 128), jnp.float32),
                   jax.ShapeDtypeStruct((N, b_dim), jnp.float32)],
    )(s0, s1, deg2, nn_bias, tw, cw, cb)


def _layer(m, c, deg2, tw, cw, cb):
    a_dim = tw.shape[0]
    b_dim = tw.shape[1]

    def body(m_ref, c_ref, d_ref, twr, cwr, cbr, a_out, c_out):
        h = jnp.where(d_ref[...] > 0, m_ref[...] + c_ref[...], 0.0)
        a = _dot(h, twr[...])
        if b_dim < 128:
            a = jnp.concatenate(
                [a, jnp.zeros((BN, 128 - b_dim), jnp.float32)], axis=1)
        a_out[...] = a
        c_out[...] = _dot(h, cwr[...]) + cbr[...]

    full = lambda shape: pl.BlockSpec(shape, lambda i: (0, 0))
    return pl.pallas_call(
        body,
        grid=(N // BN,),
        in_specs=[
            pl.BlockSpec((BN, a_dim), lambda i: (i, 0)),
            pl.BlockSpec((BN, a_dim), lambda i: (i, 0)),
            pl.BlockSpec((BN, 1), lambda i: (i, 0)),
            full((a_dim, b_dim)), full((a_dim, b_dim)), full((1, b_dim)),
        ],
        out_specs=[pl.BlockSpec((BN, 128), lambda i: (i, 0)),
                   pl.BlockSpec((BN, b_dim), lambda i: (i, 0))],
        out_shape=[jax.ShapeDtypeStruct((N, 128), jnp.float32),
                   jax.ShapeDtypeStruct((N, b_dim), jnp.float32)],
    )(m, c, deg2, tw, cw, cb)


def _final(m, c, deg2):
    b_dim = m.shape[1]

    def body(m_ref, c_ref, d_ref, out):
        out[...] = jnp.where(d_ref[...] > 0, m_ref[...] + c_ref[...], 0.0)

    return pl.pallas_call(
        body,
        grid=(N // BN,),
        in_specs=[
            pl.BlockSpec((BN, b_dim), lambda i: (i, 0)),
            pl.BlockSpec((BN, b_dim), lambda i: (i, 0)),
            pl.BlockSpec((BN, 1), lambda i: (i, 0)),
        ],
        out_specs=pl.BlockSpec((BN, b_dim), lambda i: (i, 0)),
        out_shape=jax.ShapeDtypeStruct((N, b_dim), jnp.float32),
    )(m, c, deg2)


def kernel(x, edge_index, edge_attr, params):
    p = params
    src = edge_index[0]
    dst = edge_index[1]

    def fold(i):
        s = p['mg%d' % i] * BN_SCALE
        return p['mW%d' % i] * s[None, :], (p['mb%d' % i] * s + p['mbeta%d' % i])

    w1, c1 = fold(1)
    w2, c2 = fold(2)
    w3, c3 = fold(3)
    w4, c4 = fold(4)
    ww1 = jnp.concatenate([w1, c1[None, :]], axis=0)
    ws = (ww1, w2, c2[None, :], w3, c3[None, :], w4, c4[None, :],
          p['mW5'], p['mb5'][None, :])

    ea1 = jnp.concatenate([edge_attr, jnp.ones((E, 1), jnp.float32)], axis=1)

    x_wide = jnp.pad(x, ((0, 0), (0, 121)))
    xs = _xs_gather(x_wide, src).reshape(E, 8)
    srcp, base, deg = _sort_csr(dst, src)
    msg0, msg1 = _fused_msg(ea1, xs, ws)
    s0, s1 = _seg_sum(msg0, msg1, dst, jnp.zeros((NP, 128), jnp.float32))
    s0, s1 = s0[:N], s1[:N]
    deg2 = deg[:N].reshape(N, 1)

    dims = [(256, 128), (128, 64), (64, 32), (32, 16), (16, 16)]
    a_arr = c_arr = None
    for i, (ad, bd) in enumerate(dims):
        tw = p['tW%d' % i]
        cw = p['pW%d' % i] - tw
        cb = (p['tb%d' % i] + p['pb%d' % i])[None, :]
        if i == 0:
            a_arr, c_arr = _layer0(s0, s1, deg2, p['nn_bias'][None, :], tw, cw, cb)
        else:
            a_arr, c_arr = _layer(m_arr, c_arr, deg2, tw, cw, cb)
        m_flat = _seg_max(a_arr, srcp, base, bd)
        m_arr = m_flat.reshape(NP, bd)[:N]
    return _final(m_arr, c_arr, deg2)


# speedup vs baseline: 6.5543x; 1.2125x over previous
import functools

import jax
import jax.numpy as jnp
from jax import lax
from jax.experimental import pallas as pl
from jax.experimental.pallas import tpu as pltpu
from jax.experimental.pallas import tpu_sc as plsc

N = 10000
E = 160000
NP = 10240
NC, NS = 2, 16
NW = NC * NS
EPT = E // NS
NPT = NP // NS
NPW = NP // NW
EPS = 1e-5
BN_SCALE = 1.0 / (1.0 + EPS) ** 0.5
NEG = -3.0e38

BE = 1600
BN = 1000

_SCP = pltpu.CompilerParams(needs_layout_passes=False)


def _mesh():
    return plsc.VectorSubcoreMesh(core_axis_name="c", subcore_axis_name="s",
                                  num_cores=NC, num_subcores=NS)


def _dot(a, b):
    return jnp.dot(a, b, precision=jax.lax.Precision.HIGHEST)


def _dot_hi(a, b):
    return jnp.dot(a, b, precision=jax.lax.Precision.DEFAULT)


def _mlp_msg_body(ea_ref, xs_ref,
                  w1, b1, g1, t1, w2, b2, g2, t2,
                  w3, b3, g3, t3, w4, b4, g4, t4, w5, b5,
                  out0_ref, out1_ref):
    s = jnp.sqrt(jnp.float32(1.0 + EPS))

    def lyr(h, w, bb, gg, tt):
        t = _dot_hi(h, w[...]) + bb[...]
        return jax.nn.relu(gg[...] * t / s + tt[...])

    h = lyr(ea_ref[...], w1, b1, g1, t1)
    h = lyr(h, w2, b2, g2, t2)
    h = lyr(h, w3, b3, g3, t3)
    h = lyr(h, w4, b4, g4, t4)
    xs = xs_ref[...]
    acc = jnp.zeros((BE, 256), jnp.float32)
    for i in range(7):
        z = _dot_hi(h, w5[...][:, i * 256:(i + 1) * 256]) \
            + b5[...][:, i * 256:(i + 1) * 256]
        acc = acc + xs[:, i:i + 1] * jax.nn.sigmoid(z)
    out0_ref[...] = acc[:, :128]
    out1_ref[...] = acc[:, 128:]


def _fused_msg(ea, xs, ws):
    full = lambda shape: pl.BlockSpec(shape, lambda i: (0, 0))
    dims = [(3, 256), (256, 128), (128, 64), (64, 32)]
    specs = []
    for a, bdim in dims:
        specs += [full((a, bdim)), full((1, bdim)), full((1, bdim)),
                  full((1, bdim))]
    return pl.pallas_call(
        _mlp_msg_body,
        grid=(E // BE,),
        in_specs=[
            pl.BlockSpec((BE, 3), lambda i: (i, 0)),
            pl.BlockSpec((BE, 8), lambda i: (i, 0)),
            *specs,
            full((32, 1792)), full((1, 1792)),
        ],
        out_specs=[pl.BlockSpec((BE, 128), lambda i: (i, 0)),
                   pl.BlockSpec((BE, 128), lambda i: (i, 0))],
        out_shape=[jax.ShapeDtypeStruct((E, 128), jnp.float32),
                   jax.ShapeDtypeStruct((E, 128), jnp.float32)],
    )(ea, xs, *ws)


def _xs_gather(x_wide, src):
    @functools.partial(
        pl.kernel,
        out_type=jax.ShapeDtypeStruct((E * 8,), jnp.float32),
        mesh=_mesh(),
        scratch_types=[
            pltpu.VMEM((128,), jnp.int32),
            pltpu.VMEM((128, 128), jnp.float32),
            pltpu.VMEM((5000 * 8,), jnp.float32),
            pltpu.SemaphoreType.DMA,
        ],
        compiler_params=_SCP,
    )
    def k(x_hbm, src_hbm, xs_hbm, idx_v, grows, rows_v, sem):
        w = lax.axis_index("c") * NS + lax.axis_index("s")
        e0 = w * 5000
        riota = lax.iota(jnp.int32, 16)

        def body(jj, _):
            boff = jnp.minimum(jj * 128, 5000 - 128)
            pltpu.sync_copy(src_hbm.at[pl.ds(e0 + boff, 128)], idx_v)
            pltpu.async_copy(x_hbm.at[idx_v], grows, sem).wait()
            for g in range(8):
                ridx = riota + g * 16
                for kcol in range(8):
                    vals = plsc.load_gather(
                        grows, [ridx, jnp.full((16,), kcol, jnp.int32)])
                    plsc.store_scatter(
                        rows_v, [(boff + ridx) * 8 + kcol], vals)
            return 0

        lax.fori_loop(0, 40, body, 0)
        pltpu.sync_copy(rows_v, xs_hbm.at[pl.ds(e0 * 8, 5000 * 8)])

    return k(x_wide, src)


def _sort_csr(dst, src):
    @functools.partial(
        pl.kernel,
        out_type=(
            jax.ShapeDtypeStruct((E + 128,), jnp.int32),
            jax.ShapeDtypeStruct((NP + 16,), jnp.int32),
            jax.ShapeDtypeStruct((NP,), jnp.float32),
        ),
        mesh=_mesh(),
        scratch_types=[
            pltpu.VMEM((EPT,), jnp.int32),
            pltpu.VMEM((EPT,), jnp.int32),
            pltpu.VMEM((NP,), jnp.int32),
            pltpu.VMEM((NS * NPT,), jnp.int32),
            pltpu.VMEM((NPT,), jnp.int32),
            pltpu.VMEM((NPT,), jnp.int32),
            pltpu.VMEM((NPT,), jnp.float32),
            pltpu.VMEM((NPT,), jnp.int32),
            pltpu.VMEM((16,), jnp.int32),
            pltpu.VMEM((NS * 16,), jnp.int32),
            pltpu.VMEM((128,), jnp.int32),
            pltpu.VMEM((128,), jnp.int32),
            pltpu.VMEM_SHARED((NS * NP,), jnp.int32),
            pltpu.VMEM_SHARED((NS * 16,), jnp.int32),
        ],
        compiler_params=_SCP,
    )
    def k(dst_hbm, src_hbm, srcp_hbm, base_hbm, deg_hbm,
          dstb, srcb, hist_v, gcols, tot_v, base_v, deg_v, row_v, tbuf,
          sums2, posb, srow, g_sp, sums_sp):
        cid = lax.axis_index("c")
        t = lax.axis_index("s")

        @pl.when(cid == 0)
        def _():
            pltpu.sync_copy(dst_hbm.at[pl.ds(t * EPT, EPT)], dstb)

            def zero(i, _):
                hist_v[pl.ds(i * 16, 16)] = jnp.zeros((16,), jnp.int32)
                return 0
            lax.fori_loop(0, NP // 16, zero, 0)

            def hbody(i, _):
                d = dstb[pl.ds(i * 16, 16)]
                occ, last = plsc.scan_count(d)
                plsc.addupdate_scatter(hist_v, [d], occ, mask=last)
                return 0
            lax.fori_loop(0, EPT // 16, hbody, 0)
            pltpu.sync_copy(hist_v, g_sp.at[pl.ds(t * NP, NP)])
            plsc.subcore_barrier()

            for r in range(NS):
                pltpu.sync_copy(g_sp.at[pl.ds(r * NP + t * NPT, NPT)],
                                gcols.at[pl.ds(r * NPT, NPT)])
            def tbody(i, _):
                s = jnp.zeros((16,), jnp.int32)
                for r in range(NS):
                    s = s + gcols[pl.ds(r * NPT + i * 16, 16)]
                tot_v[pl.ds(i * 16, 16)] = s
                return 0
            lax.fori_loop(0, NPT // 16, tbody, 0)

            def sbody(i, acc):
                return acc + tot_v[pl.ds(i * 16, 16)]
            rs = lax.fori_loop(0, NPT // 16, sbody, jnp.zeros((16,), jnp.int32))
            range_sum = jnp.sum(rs)
            tbuf[...] = jnp.full((16,), 0, jnp.int32) + range_sum
            pltpu.sync_copy(tbuf, sums_sp.at[pl.ds(t * 16, 16)])
            plsc.subcore_barrier()

            pltpu.sync_copy(sums_sp, sums2)
            lanes = lax.iota(jnp.int32, 16)
            svec = plsc.load_gather(sums2, [lanes * 16])
            base0 = jnp.sum(jnp.where(lanes < t, svec, 0))

            def cbody(i, carry):
                v = tot_v[pl.ds(i * 16, 16)]
                inc = plsc.cumsum(v)
                base_v[pl.ds(i * 16, 16)] = carry + inc - v
                deg_v[pl.ds(i * 16, 16)] = v.astype(jnp.float32)
                return carry + jnp.max(inc)
            lax.fori_loop(0, NPT // 16, cbody, base0)

            pltpu.sync_copy(base_v, base_hbm.at[pl.ds(t * NPT, NPT)])
            pltpu.sync_copy(deg_v, deg_hbm.at[pl.ds(t * NPT, NPT)])

            @pl.when(t == NS - 1)
            def _():
                tbuf[...] = jnp.full((16,), E, jnp.int32)
                pltpu.sync_copy(tbuf, base_hbm.at[pl.ds(NP, 16)])

            def copy_base(i, _):
                row_v[pl.ds(i * 16, 16)] = base_v[pl.ds(i * 16, 16)]
                return 0
            lax.fori_loop(0, NPT // 16, copy_base, 0)
            for r in range(NS):
                pltpu.sync_copy(row_v, g_sp.at[pl.ds(r * NP + t * NPT, NPT)])
                def add_r(i, _):
                    row_v[pl.ds(i * 16, 16)] = (
                        row_v[pl.ds(i * 16, 16)]
                        + gcols[pl.ds(r * NPT + i * 16, 16)])
                    return 0
                if r < NS - 1:
                    lax.fori_loop(0, NPT // 16, add_r, 0)
            plsc.subcore_barrier()

            pltpu.sync_copy(g_sp.at[pl.ds(t * NP, NP)], hist_v)
            pltpu.sync_copy(src_hbm.at[pl.ds(t * EPT, EPT)], srcb)

            def chunk(j, _):
                for g in range(8):
                    d = dstb[pl.ds(j * 128 + g * 16, 16)]
                    occ, last = plsc.scan_count(d)
                    cur = plsc.load_gather(hist_v, [d])
                    posb[pl.ds(g * 16, 16)] = cur + occ - 1
                    plsc.addupdate_scatter(hist_v, [d], occ, mask=last)
                    srow[pl.ds(g * 16, 16)] = srcb[pl.ds(j * 128 + g * 16, 16)]
                pltpu.sync_copy(srow, srcp_hbm.at[posb])
                return 0
            lax.fori_loop(0, EPT // 128, chunk, 0)

            if EPT % 128:
                j0 = (EPT // 128) * 128
                d = dstb[pl.ds(j0, 16)]
                occ, last = plsc.scan_count(d)
                cur = plsc.load_gather(hist_v, [d])
                posb[pl.ds(0, 16)] = cur + occ - 1
                plsc.addupdate_scatter(hist_v, [d], occ, mask=last)
                srow[pl.ds(0, 16)] = srcb[pl.ds(j0, 16)]
                def padp(g, _):
                    posb[pl.ds(16 + g * 16, 16)] = jnp.full((16,), E, jnp.int32)
                    return 0
                lax.fori_loop(0, 7, padp, 0)
                pltpu.sync_copy(srow, srcp_hbm.at[posb])

    return k(dst, src)


def _seg_sum(msg0, msg1, dst, zeros_np):
    @functools.partial(
        pl.kernel,
        out_type=(jax.ShapeDtypeStruct((NP, 128), jnp.float32),
                  jax.ShapeDtypeStruct((NP, 128), jnp.float32)),
        mesh=_mesh(),
        scratch_types=[
            pltpu.VMEM((128, 128), jnp.float32),
            pltpu.VMEM((128,), jnp.int32),
            pltpu.VMEM((16,), jnp.int32),
            pltpu.VMEM((128, 128), jnp.float32),
            pltpu.VMEM_SHARED((NP, 128), jnp.float32),
        ],
        compiler_params=_SCP,
    )
    def k(m0_hbm, m1_hbm, dst_hbm, z_hbm, out0_hbm, out1_hbm,
          mbuf, didx, didx16, wb, s_sp):
        cid = lax.axis_index("c")
        t = lax.axis_index("s")
        m_hbm = [m0_hbm, m1_hbm]
        out_hbm = [out0_hbm, out1_hbm]

        @pl.when(t == 0)
        def _():
            pltpu.sync_copy(z_hbm, s_sp)
        plsc.subcore_barrier()

        for c in range(NC):
            @pl.when(cid == c)
            def _(c=c):
                def body(j, _):
                    e0 = t * EPT + j * 128
                    pltpu.sync_copy(m_hbm[c].at[pl.ds(e0, 128), :], mbuf)
                    pltpu.sync_copy(dst_hbm.at[pl.ds(e0, 128)], didx)
                    pltpu.sync_copy(mbuf, s_sp.at[didx], add=True)
                    return 0
                lax.fori_loop(0, EPT // 128, body, 0)
                e0t = t * EPT + (EPT // 128) * 128
                pltpu.sync_copy(m_hbm[c].at[pl.ds(e0t, 16), :],
                                mbuf.at[pl.ds(0, 16), :])
                pltpu.sync_copy(dst_hbm.at[pl.ds(e0t, 16)], didx16)
                pltpu.sync_copy(mbuf.at[pl.ds(0, 16), :],
                                s_sp.at[didx16], add=True)
        plsc.subcore_barrier()

        n0 = t * 640
        iota16 = lax.iota(jnp.int32, 16)
        def wb_chunk(kc, _):
            r0 = n0 + kc * 128
            for g in range(8):
                didx[pl.ds(g * 16, 16)] = r0 + g * 16 + iota16
            pltpu.sync_copy(s_sp.at[didx], wb)
            for c in range(NC):
                @pl.when(cid == c)
                def _(c=c):
                    pltpu.sync_copy(wb, out_hbm[c].at[pl.ds(r0, 128), :])
            return 0
        lax.fori_loop(0, 5, wb_chunk, 0)

    return k(msg0, msg1, dst, zeros_np)


def _seg_max(a, srcp, base, b):
    nb = b // 16

    @functools.partial(
        pl.kernel,
        out_type=jax.ShapeDtypeStruct((NP * b,), jnp.float32),
        mesh=_mesh(),
        scratch_types=[
            pltpu.VMEM((336,), jnp.int32),
            pltpu.VMEM((128,), jnp.int32),
            pltpu.VMEM((128,), jnp.int32),
            pltpu.VMEM((128, 128), jnp.float32),
            pltpu.VMEM((128, 128), jnp.float32),
            pltpu.VMEM((NPW * b,), jnp.float32),
            pltpu.SemaphoreType.DMA,
            pltpu.SemaphoreType.DMA,
        ],
        compiler_params=_SCP,
    )
    def k(a_hbm, srcp_hbm, base_hbm, m_hbm, base_sm, sidx0, sidx1,
          rows0, rows1, outb, sem0, sem1):
        w = lax.axis_index("c") * NS + lax.axis_index("s")
        n0 = w * NPW

        def bval(i):
            return base_sm[pl.ds(i, 16)][0]

        pltpu.sync_copy(base_hbm.at[pl.ds(n0, 328)],
                        base_sm.at[pl.ds(0, 328)])
        sp = bval(0)
        ep = bval(NPW)

        def zero(i, _):
            outb[pl.ds(i * 16, 16)] = jnp.full((16,), NEG, jnp.float32)
            return 0
        lax.fori_loop(0, NPW * b // 16, zero, 0)

        neg = tuple(jnp.full((16,), NEG, jnp.float32) for _ in range(nb))
        iota16 = lax.iota(jnp.int32, 16)
        cols = [iota16 + q * 16 for q in range(nb)]

        def issue(cg, sidx_s, rows_s, sem_s):
            pltpu.sync_copy(srcp_hbm.at[pl.ds(cg * 128, 128)], sidx_s)
            pltpu.async_copy(a_hbm.at[sidx_s], rows_s, sem_s)

        def consume(carry, cg, rows_s):
            n, p, acc0 = carry
            hi = jnp.minimum(ep, (cg + 1) * 128)
            clo = cg * 128

            def cond(st):
                return st[1] < hi

            def step(st):
                n_, p_, acc = st
                e_n = bval(n_ - n0 + 1)
                seg_hi = jnp.minimum(e_n, hi)

                def pos_body(pp, acc_):
                    rvec = jnp.full((16,), pp - clo, jnp.int32)
                    return tuple(
                        jnp.maximum(acc_[q],
                                    plsc.load_gather(rows_s, [rvec, cols[q]]))
                        for q in range(nb))
                acc = lax.fori_loop(p_, seg_hi, pos_body, acc)

                flushed = e_n <= hi

                @pl.when(flushed)
                def _():
                    o = (n_ - n0) * b
                    for q in range(nb):
                        outb[pl.ds(o + q * 16, 16)] = acc[q]

                acc = tuple(
                    jnp.where(flushed, neg[q], acc[q]) for q in range(nb))
                n_ = jnp.where(flushed, n_ + 1, n_)
                return (n_, seg_hi, acc)

            return lax.while_loop(cond, step, (n, jnp.maximum(p, clo), acc0))

        def wait0():
            pltpu.make_async_copy(a_hbm.at[sidx0], rows0, sem0).wait()

        def wait1():
            pltpu.make_async_copy(a_hbm.at[sidx1], rows1, sem1).wait()

        cg0 = sp // 128
        cg1 = (ep + 127) // 128

        @pl.when(cg0 < cg1)
        def _():
            issue(cg0, sidx0, rows0, sem0)

        def pair(i, carry):
            cga = cg0 + 2 * i
            cgb = cga + 1
            issue(cgb, sidx1, rows1, sem1)
            wait0()
            carry = consume(carry, cga, rows0)

            @pl.when(cgb + 1 < cg1)
            def _():
                issue(cgb + 1, sidx0, rows0, sem0)

            wait1()
            carry = consume(carry, cgb, rows1)
            return carry

        carry = lax.fori_loop(0, (cg1 - cg0) // 2, pair, (n0, sp, neg))

        def tail(c):
            wait0()
            return consume(c, cg1 - 1, rows0)

        odd = jnp.logical_and(cg0 < cg1, (cg1 - cg0) % 2 == 1)
        lax.cond(odd, tail, lambda c: c, carry)

        pltpu.sync_copy(outb, m_hbm.at[pl.ds(n0 * b, NPW * b)])

    return k(a, srcp, base)


def _layer0(s0, s1, deg2, nn_bias, tw, cw, cb):
    a_dim = 256
    b_dim = tw.shape[1]

    def body(s0_ref, s1_ref, d_ref, bias, twr, cwr, cbr, a_out, c_out):
        d = d_ref[...]
        s = jnp.concatenate([s0_ref[...], s1_ref[...]], axis=1)
        h = s / jnp.maximum(d, 1.0) + bias[...]
        a = _dot(h, twr[...])
        if b_dim < 128:
            a = jnp.concatenate(
                [a, jnp.zeros((BN, 128 - b_dim), jnp.float32)], axis=1)
        a_out[...] = a
        c_out[...] = _dot(h, cwr[...]) + cbr[...]

    full = lambda shape: pl.BlockSpec(shape, lambda i: (0, 0))
    return pl.pallas_call(
        body,
        grid=(N // BN,),
        in_specs=[
            pl.BlockSpec((BN, 128), lambda i: (i, 0)),
            pl.BlockSpec((BN, 128), lambda i: (i, 0)),
            pl.BlockSpec((BN, 1), lambda i: (i, 0)),
            full((1, a_dim)), full((a_dim, b_dim)), full((a_dim, b_dim)),
            full((1, b_dim)),
        ],
        out_specs=[pl.BlockSpec((BN, 128), lambda i: (i, 0)),
                   pl.BlockSpec((BN, b_dim), lambda i: (i, 0))],
        out_shape=[jax.ShapeDtypeStruct((N, 128), jnp.float32),
                   jax.ShapeDtypeStruct((N, b_dim), jnp.float32)],
    )(s0, s1, deg2, nn_bias, tw, cw, cb)


def _layer(m, c, deg2, tw, cw, cb):
    a_dim = tw.shape[0]
    b_dim = tw.shape[1]

    def body(m_ref, c_ref, d_ref, twr, cwr, cbr, a_out, c_out):
        h = jnp.where(d_ref[...] > 0, m_ref[...] + c_ref[...], 0.0)
        a = _dot(h, twr[...])
        if b_dim < 128:
            a = jnp.concatenate(
                [a, jnp.zeros((BN, 128 - b_dim), jnp.float32)], axis=1)
        a_out[...] = a
        c_out[...] = _dot(h, cwr[...]) + cbr[...]

    full = lambda shape: pl.BlockSpec(shape, lambda i: (0, 0))
    return pl.pallas_call(
        body,
        grid=(N // BN,),
        in_specs=[
            pl.BlockSpec((BN, a_dim), lambda i: (i, 0)),
            pl.BlockSpec((BN, a_dim), lambda i: (i, 0)),
            pl.BlockSpec((BN, 1), lambda i: (i, 0)),
            full((a_dim, b_dim)), full((a_dim, b_dim)), full((1, b_dim)),
        ],
        out_specs=[pl.BlockSpec((BN, 128), lambda i: (i, 0)),
                   pl.BlockSpec((BN, b_dim), lambda i: (i, 0))],
        out_shape=[jax.ShapeDtypeStruct((N, 128), jnp.float32),
                   jax.ShapeDtypeStruct((N, b_dim), jnp.float32)],
    )(m, c, deg2, tw, cw, cb)


def _final(m, c, deg2):
    b_dim = m.shape[1]

    def body(m_ref, c_ref, d_ref, out):
        out[...] = jnp.where(d_ref[...] > 0, m_ref[...] + c_ref[...], 0.0)

    return pl.pallas_call(
        body,
        grid=(N // BN,),
        in_specs=[
            pl.BlockSpec((BN, b_dim), lambda i: (i, 0)),
            pl.BlockSpec((BN, b_dim), lambda i: (i, 0)),
            pl.BlockSpec((BN, 1), lambda i: (i, 0)),
        ],
        out_specs=pl.BlockSpec((BN, b_dim), lambda i: (i, 0)),
        out_shape=jax.ShapeDtypeStruct((N, b_dim), jnp.float32),
    )(m, c, deg2)


def kernel(x, edge_index, edge_attr, params):
    p = params
    src = edge_index[0]
    dst = edge_index[1]

    ws = []
    for i in range(1, 5):
        ws += [p['mW%d' % i], p['mb%d' % i][None, :],
               p['mg%d' % i][None, :], p['mbeta%d' % i][None, :]]
    ws += [p['mW5'], p['mb5'][None, :]]

    x_wide = jnp.pad(x, ((0, 0), (0, 121)))
    xs = _xs_gather(x_wide, src).reshape(E, 8)
    srcp, base, deg = _sort_csr(dst, src)
    msg0, msg1 = _fused_msg(edge_attr, xs, ws)
    s0, s1 = _seg_sum(msg0, msg1, dst, jnp.zeros((NP, 128), jnp.float32))
    s0, s1 = s0[:N], s1[:N]
    deg2 = deg[:N].reshape(N, 1)

    dims = [(256, 128), (128, 64), (64, 32), (32, 16), (16, 16)]
    a_arr = c_arr = None
    for i, (ad, bd) in enumerate(dims):
        tw = p['tW%d' % i]
        cw = p['pW%d' % i] - tw
        cb = (p['tb%d' % i] + p['pb%d' % i])[None, :]
        if i == 0:
            a_arr, c_arr = _layer0(s0, s1, deg2, p['nn_bias'][None, :], tw, cw, cb)
        else:
            a_arr, c_arr = _layer(m_arr, c_arr, deg2, tw, cw, cb)
        m_flat = _seg_max(a_arr, srcp, base, bd)
        m_arr = m_flat.reshape(NP, bd)[:N]
    return _final(m_arr, c_arr, deg2)
